# baseline (speedup 1.0000x reference)
import functools

import jax
import jax.numpy as jnp
from jax.experimental import pallas as pl
from jax.experimental.pallas import tpu as pltpu

_CHUNK = 25600
_GROUPS = _CHUNK // 128
_NLOG2E = -1.4426950408889634
_NLN2 = -0.6931471805599453


def _body(logits_ref, noise_ref, idx_ref, logp_ref, cnt_ref,
          acc_m, acc_gid, acc_t, acc_cnt, *, V):
    step = pl.program_id(0)
    nsteps = pl.num_programs(0)
    B = logits_ref.shape[0]

    @pl.when(step == 0)
    def _init():
        acc_m[...] = jnp.full(acc_m.shape, -1.0, acc_m.dtype)
        acc_gid[...] = jnp.zeros(acc_gid.shape, acc_gid.dtype)
        acc_t[...] = jnp.zeros(acc_t.shape, acc_t.dtype)
        acc_cnt[...] = jnp.zeros(acc_cnt.shape, acc_cnt.dtype)

    lane = jax.lax.broadcasted_iota(jnp.int32, (B, 128), 1)
    base = step * _CHUNK

    def _group(j, carry):
        m, gid, tb, cnt = carry
        x = logits_ref[:, pl.ds(j * 128, 128)]
        u = noise_ref[:, pl.ds(j * 128, 128)]
        t = jnp.exp2(x * _NLOG2E)
        a = 0.0005 * t + 0.9995
        w = jnp.log2(u) * _NLN2
        onep = 1.0 + t
        r = a / (onep * w)
        valid = lane < (V - base - j * 128)
        r = jnp.where(valid, r, -1.0)
        gate = (u * onep < a) & valid
        cnt = cnt + jnp.where(gate, 1.0, 0.0)
        better = r > m
        m = jnp.where(better, r, m)
        gid = jnp.where(better, jnp.full((B, 128), step * _GROUPS + j,
                                         jnp.int32), gid)
        tb = jnp.where(better, t, tb)
        return m, gid, tb, cnt

    carry = (acc_m[...], acc_gid[...], acc_t[...], acc_cnt[...])
    m, gid, tb, cnt = jax.lax.fori_loop(0, _GROUPS, _group, carry)
    acc_m[...] = m
    acc_gid[...] = gid
    acc_t[...] = tb
    acc_cnt[...] = cnt

    @pl.when(step == nsteps - 1)
    def _fin():
        mrow = jnp.max(m, axis=1, keepdims=True)
        hit = m == mrow
        eidx = gid * 128 + lane
        idx_ref[...] = jnp.min(jnp.where(hit, eidx, V), axis=1,
                               keepdims=True)
        t_at = jnp.max(jnp.where(hit, tb, -1.0), axis=1, keepdims=True)
        p = 0.999 / (1.0 + t_at) + 0.0005
        logp_ref[...] = jnp.log(p)
        cnt_ref[...] = jnp.sum(cnt, axis=1, keepdims=True)


def kernel(logits, noise):
    B, V = logits.shape
    grid = ((V + _CHUNK - 1) // _CHUNK,)
    out_shape = [
        jax.ShapeDtypeStruct((B, 1), jnp.int32),
        jax.ShapeDtypeStruct((B, 1), jnp.float32),
        jax.ShapeDtypeStruct((B, 1), jnp.float32),
    ]
    idx, chosen_logp, cnt = pl.pallas_call(
        functools.partial(_body, V=V),
        grid=grid,
        in_specs=[
            pl.BlockSpec((B, _CHUNK), lambda i: (0, i)),
            pl.BlockSpec((B, _CHUNK), lambda i: (0, i)),
        ],
        out_specs=[
            pl.BlockSpec((B, 1), lambda i: (0, 0)),
            pl.BlockSpec((B, 1), lambda i: (0, 0)),
            pl.BlockSpec((B, 1), lambda i: (0, 0)),
        ],
        out_shape=out_shape,
        scratch_shapes=[
            pltpu.VMEM((B, 128), jnp.float32),
            pltpu.VMEM((B, 128), jnp.int32),
            pltpu.VMEM((B, 128), jnp.float32),
            pltpu.VMEM((B, 128), jnp.float32),
        ],
        compiler_params=pltpu.CompilerParams(
            dimension_semantics=("arbitrary",)),
    )(logits, noise)
    return (idx[:, 0], chosen_logp[:, 0], cnt[:, 0])


# speedup vs baseline: 1.4710x; 1.4710x over previous
import functools

import jax
import jax.numpy as jnp
from jax.experimental import pallas as pl
from jax.experimental.pallas import tpu as pltpu

_CHUNK = 25600
_NLOG2E = -1.4426950408889634
_NLN2 = -0.6931471805599453


def _body(logits_ref, noise_ref, idx_ref, logp_ref, cnt_ref,
          best_m, best_idx, best_t, cnt_acc, col_s, *, V):
    step = pl.program_id(0)
    nsteps = pl.num_programs(0)

    @pl.when(step == 0)
    def _init():
        best_m[...] = jnp.full(best_m.shape, -1.0, best_m.dtype)
        best_idx[...] = jnp.zeros(best_idx.shape, best_idx.dtype)
        best_t[...] = jnp.ones(best_t.shape, best_t.dtype)
        cnt_acc[...] = jnp.zeros(cnt_acc.shape, cnt_acc.dtype)
        col_s[...] = jax.lax.broadcasted_iota(jnp.int32, col_s.shape, 1)

    x = logits_ref[...]
    u = noise_ref[...]
    t = jnp.exp2(x * _NLOG2E)
    a = 0.0005 * t + 0.9995
    w = jnp.log2(u) * _NLN2
    onep = 1.0 + t
    r = a / (onep * w)
    col = col_s[...]
    valid = col < (V - step * _CHUNK)
    r = jnp.where(valid, r, -1.0)
    gate = (u * onep < a) & valid

    m = jnp.max(r, axis=1, keepdims=True)
    hit = r == m
    lidx = jnp.min(jnp.where(hit, col, _CHUNK), axis=1, keepdims=True)
    t_at = jnp.max(jnp.where(hit, t, -1.0), axis=1, keepdims=True)
    cnt = jnp.sum(jnp.where(gate, 1.0, 0.0), axis=1, keepdims=True)

    better = m > best_m[...]
    best_idx[...] = jnp.where(better, lidx + step * _CHUNK, best_idx[...])
    best_t[...] = jnp.where(better, t_at, best_t[...])
    best_m[...] = jnp.maximum(best_m[...], m)
    cnt_acc[...] = cnt_acc[...] + cnt

    @pl.when(step == nsteps - 1)
    def _fin():
        idx_ref[...] = best_idx[...]
        p = 0.999 / (1.0 + best_t[...]) + 0.0005
        logp_ref[...] = jnp.log(p)
        cnt_ref[...] = cnt_acc[...]


def kernel(logits, noise):
    B, V = logits.shape
    grid = ((V + _CHUNK - 1) // _CHUNK,)
    out_shape = [
        jax.ShapeDtypeStruct((B, 1), jnp.int32),
        jax.ShapeDtypeStruct((B, 1), jnp.float32),
        jax.ShapeDtypeStruct((B, 1), jnp.float32),
    ]
    idx, chosen_logp, cnt = pl.pallas_call(
        functools.partial(_body, V=V),
        grid=grid,
        in_specs=[
            pl.BlockSpec((B, _CHUNK), lambda i: (0, i)),
            pl.BlockSpec((B, _CHUNK), lambda i: (0, i)),
        ],
        out_specs=[
            pl.BlockSpec((B, 1), lambda i: (0, 0)),
            pl.BlockSpec((B, 1), lambda i: (0, 0)),
            pl.BlockSpec((B, 1), lambda i: (0, 0)),
        ],
        out_shape=out_shape,
        scratch_shapes=[
            pltpu.VMEM((B, 1), jnp.float32),
            pltpu.VMEM((B, 1), jnp.int32),
            pltpu.VMEM((B, 1), jnp.float32),
            pltpu.VMEM((B, 1), jnp.float32),
            pltpu.VMEM((B, _CHUNK), jnp.int32),
        ],
        compiler_params=pltpu.CompilerParams(
            dimension_semantics=("arbitrary",)),
    )(logits, noise)
    return (idx[:, 0], chosen_logp[:, 0], cnt[:, 0])
